# Initial kernel scaffold; baseline (speedup 1.0000x reference)
import jax
import jax.numpy as jnp
from jax.experimental import pallas as pl


def kernel(tokens, pe):
    raise NotImplementedError("write your pallas kernel here")


# speedup vs baseline: 17.3399x; 17.3399x over previous
import functools

import jax
import jax.numpy as jnp
from jax import lax
from jax.experimental import pallas as pl
from jax.experimental.pallas import tpu as pltpu
from jax.experimental.pallas import tpu_sc as plsc

_NUM_CORES = 2
_NUM_SUBCORES = 16
_NUM_WORKERS = _NUM_CORES * _NUM_SUBCORES


def kernel(tokens, pe):
    B, L = tokens.shape
    _, D = pe.shape

    rows_per_worker = B // _NUM_WORKERS
    RB = 4
    n_dma = rows_per_worker // RB

    mesh = plsc.VectorSubcoreMesh(core_axis_name="c", subcore_axis_name="s")

    @functools.partial(
        pl.kernel,
        out_type=jax.ShapeDtypeStruct((B, L, D), jnp.float32),
        mesh=mesh,
        scratch_types=[
            pltpu.VMEM((RB, L, D), jnp.float32),
            pltpu.SemaphoreType.DMA,
        ],
    )
    def pe_broadcast(pe_hbm, out_hbm, rep_v, sem):
        wid = lax.axis_index("s") * _NUM_CORES + lax.axis_index("c")
        base = wid * rows_per_worker
        for j in range(RB):
            pltpu.sync_copy(pe_hbm.at[pl.ds(0, L)], rep_v.at[j])
        for i in range(n_dma):
            pltpu.make_async_copy(
                rep_v, out_hbm.at[pl.ds(base + i * RB, RB)], sem
            ).start()
        for i in range(n_dma):
            pltpu.make_async_copy(
                rep_v, out_hbm.at[pl.ds(base + i * RB, RB)], sem
            ).wait()

    return pe_broadcast(pe)


# speedup vs baseline: 17.4897x; 1.0086x over previous
import functools

import jax
import jax.numpy as jnp
from jax import lax
from jax.experimental import pallas as pl
from jax.experimental.pallas import tpu as pltpu
from jax.experimental.pallas import tpu_sc as plsc

_NUM_CORES = 2
_NUM_SUBCORES = 16
_NUM_WORKERS = _NUM_CORES * _NUM_SUBCORES


def kernel(tokens, pe):
    B, L = tokens.shape
    _, D = pe.shape

    rows_per_worker = B // _NUM_WORKERS
    RB = 4
    n_dma = rows_per_worker // RB

    mesh = plsc.VectorSubcoreMesh(core_axis_name="c", subcore_axis_name="s")

    @functools.partial(
        pl.kernel,
        out_type=jax.ShapeDtypeStruct((B, L, D), jnp.float32),
        mesh=mesh,
        scratch_types=[
            pltpu.VMEM((RB, L, D), jnp.float32),
            pltpu.SemaphoreType.DMA,
        ],
    )
    def pe_broadcast(pe_hbm, out_hbm, rep_v, sem):
        wid = lax.axis_index("s") * _NUM_CORES + lax.axis_index("c")
        base = wid * rows_per_worker
        for j in range(RB):
            pltpu.make_async_copy(pe_hbm.at[pl.ds(0, L)], rep_v.at[j], sem).start()
        for j in range(RB):
            pltpu.make_async_copy(pe_hbm.at[pl.ds(0, L)], rep_v.at[j], sem).wait()
        for i in range(n_dma):
            pltpu.make_async_copy(
                rep_v, out_hbm.at[pl.ds(base + i * RB, RB)], sem
            ).start()
        for i in range(n_dma):
            pltpu.make_async_copy(
                rep_v, out_hbm.at[pl.ds(base + i * RB, RB)], sem
            ).wait()

    return pe_broadcast(pe)
